# baseline (speedup 1.0000x reference)
import dataclasses
import functools

import jax
import jax.numpy as jnp
from jax import lax
from jax.experimental import pallas as pl
from jax.experimental.pallas import tpu as pltpu
from jax.experimental.pallas import tpu_sc as plsc

NUM_EMOTIONS = 1000
EMO_DIM = 64
BATCH = 16384

_NC = 2
_NS = 16
_L = 16

_NBG = 2
_NDG = 8
_DPG = EMO_DIM // _NDG
_GP = _DPG + 1
_BPS = BATCH // _NC // _NBG
_JPS = _BPS // 128

_mesh = plsc.VectorSubcoreMesh(core_axis_name="c", subcore_axis_name="s")

_cp = pltpu.CompilerParams()
if "needs_layout_passes" in pltpu.CompilerParams.__dataclass_fields__:
    _cp = dataclasses.replace(_cp, needs_layout_passes=False)
if "use_tc_tiling_on_sc" in pltpu.CompilerParams.__dataclass_fields__:
    _cp = dataclasses.replace(_cp, use_tc_tiling_on_sc=False)


@jax.jit
def _emotion_encode(emo_id, strength, table_grouped):
    @functools.partial(
        pl.kernel,
        out_type=jax.ShapeDtypeStruct((BATCH // 2, 2 * EMO_DIM), jnp.float32),
        mesh=_mesh,
        compiler_params=_cp,
        scratch_types=[
            pltpu.VMEM((NUM_EMOTIONS, _GP), jnp.float32),
            pltpu.VMEM((_BPS,), jnp.int32),
            pltpu.VMEM((_BPS,), jnp.float32),
        ]
        + [pltpu.VMEM((_DPG, 128), jnp.float32) for _ in range(2)]
        + [pltpu.SemaphoreType.DMA],
    )
    def k(emo_hbm, str_hbm, tab_hbm, out_hbm, tab_v, idx_v, str_v,
          t0, t1, sem_o):
        sid = lax.axis_index("s")
        cid = lax.axis_index("c")
        g = sid // _NDG
        h = sid % _NDG
        base = cid * (BATCH // _NC) + g * _BPS

        pltpu.sync_copy(
            tab_hbm.at[:, pl.ds(h * _DPG, _DPG)],
            tab_v.at[:, pl.ds(0, _DPG)],
        )
        pltpu.sync_copy(emo_hbm.at[pl.ds(base, _BPS)], idx_v)
        pltpu.sync_copy(str_hbm.at[pl.ds(base, _BPS)], str_v)

        jg0 = base // 128

        @pl.loop(0, _JPS // 2)
        def _(p):
            for half, tv in enumerate((t0, t1)):
                jl = 2 * p + half

                @pl.when(p > 0)
                def _(tv=tv):
                    pltpu.make_async_copy(
                        tv, out_hbm.at[pl.ds(0, _DPG)], sem_o
                    ).wait()

                @plsc.parallel_loop(0, 128 // _L)
                def _(bg, jl=jl, tv=tv):
                    o = jl * 128 + bg * _L
                    e = idx_v[pl.ds(o, _L)]
                    s = str_v[pl.ds(o, _L)]
                    zero = jnp.zeros((_L,), jnp.int32)

                    @plsc.parallel_loop(0, _DPG, unroll=8)
                    def _(r):
                        v = plsc.load_gather(tab_v, [e, zero + r])
                        tv[r, pl.ds(bg * _L, _L)] = v * s

                jg = jg0 + jl
                pltpu.async_copy(
                    tv, out_hbm.at[pl.ds((h * 128 + jg) * _DPG, _DPG)], sem_o
                )

        for tv in (t0, t1):
            pltpu.make_async_copy(
                tv, out_hbm.at[pl.ds(0, _DPG)], sem_o
            ).wait()

    return k(emo_id, strength, table_grouped)


def kernel(emo_id, strength, table):
    w = _emotion_encode(emo_id.astype(jnp.int32), strength, table)
    return (
        w.reshape(_NDG, 128, _DPG, 128)
        .transpose(1, 3, 0, 2)
        .reshape(BATCH, EMO_DIM)
    )


# speedup vs baseline: 1.0515x; 1.0515x over previous
import dataclasses
import functools

import jax
import jax.numpy as jnp
from jax import lax
from jax.experimental import pallas as pl
from jax.experimental.pallas import tpu as pltpu
from jax.experimental.pallas import tpu_sc as plsc

NUM_EMOTIONS = 1000
EMO_DIM = 64
BATCH = 16384

_NC = 2
_NS = 16
_L = 16

_NBG = 2
_NDG = 8
_DPG = EMO_DIM // _NDG
_GP = _DPG + 1
_BPS = BATCH // _NC // _NBG
_JPS = _BPS // 128

_mesh = plsc.VectorSubcoreMesh(core_axis_name="c", subcore_axis_name="s")

_cp = pltpu.CompilerParams()
if "needs_layout_passes" in pltpu.CompilerParams.__dataclass_fields__:
    _cp = dataclasses.replace(_cp, needs_layout_passes=False)
if "use_tc_tiling_on_sc" in pltpu.CompilerParams.__dataclass_fields__:
    _cp = dataclasses.replace(_cp, use_tc_tiling_on_sc=False)


@jax.jit
def _emotion_encode(emo_id, strength, table_grouped):
    @functools.partial(
        pl.kernel,
        out_type=jax.ShapeDtypeStruct((BATCH // 2, 2 * EMO_DIM), jnp.float32),
        mesh=_mesh,
        compiler_params=_cp,
        scratch_types=[
            pltpu.VMEM_SHARED((NUM_EMOTIONS, EMO_DIM), jnp.float32),
            pltpu.VMEM((NUM_EMOTIONS, _GP), jnp.float32),
            pltpu.VMEM((_BPS,), jnp.int32),
            pltpu.VMEM((_BPS,), jnp.float32),
        ]
        + [pltpu.VMEM((_DPG, 128), jnp.float32) for _ in range(2)]
        + [pltpu.SemaphoreType.DMA],
    )
    def k(emo_hbm, str_hbm, tab_hbm, out_hbm, tab_s, tab_v, idx_v, str_v,
          t0, t1, sem_o):
        sid = lax.axis_index("s")
        cid = lax.axis_index("c")
        g = sid // _NDG
        h = sid % _NDG
        base = cid * (BATCH // _NC) + g * _BPS

        @pl.when(sid == 0)
        def _():
            pltpu.sync_copy(tab_hbm, tab_s)

        pltpu.sync_copy(emo_hbm.at[pl.ds(base, _BPS)], idx_v)
        pltpu.sync_copy(str_hbm.at[pl.ds(base, _BPS)], str_v)
        plsc.subcore_barrier()
        pltpu.sync_copy(
            tab_s.at[:, pl.ds(h * _DPG, _DPG)],
            tab_v.at[:, pl.ds(0, _DPG)],
        )

        jg0 = base // 128

        @pl.loop(0, _JPS // 2)
        def _(p):
            for half, tv in enumerate((t0, t1)):
                jl = 2 * p + half

                @pl.when(p > 0)
                def _(tv=tv):
                    pltpu.make_async_copy(
                        tv, out_hbm.at[pl.ds(0, _DPG)], sem_o
                    ).wait()

                @plsc.parallel_loop(0, 128 // _L)
                def _(bg, jl=jl, tv=tv):
                    o = jl * 128 + bg * _L
                    e = idx_v[pl.ds(o, _L)]
                    s = str_v[pl.ds(o, _L)]
                    zero = jnp.zeros((_L,), jnp.int32)

                    @plsc.parallel_loop(0, _DPG, unroll=8)
                    def _(r):
                        v = plsc.load_gather(tab_v, [e, zero + r])
                        tv[r, pl.ds(bg * _L, _L)] = v * s

                jg = jg0 + jl
                pltpu.async_copy(
                    tv, out_hbm.at[pl.ds((h * 128 + jg) * _DPG, _DPG)], sem_o
                )

        for tv in (t0, t1):
            pltpu.make_async_copy(
                tv, out_hbm.at[pl.ds(0, _DPG)], sem_o
            ).wait()

    return k(emo_id, strength, table_grouped)


def kernel(emo_id, strength, table):
    w = _emotion_encode(emo_id.astype(jnp.int32), strength, table)
    return (
        w.reshape(_NDG, 128, _DPG, 128)
        .transpose(1, 3, 0, 2)
        .reshape(BATCH, EMO_DIM)
    )


# speedup vs baseline: 1.2572x; 1.1956x over previous
import dataclasses
import functools

import jax
import jax.numpy as jnp
from jax import lax
from jax.experimental import pallas as pl
from jax.experimental.pallas import tpu as pltpu
from jax.experimental.pallas import tpu_sc as plsc

NUM_EMOTIONS = 1000
EMO_DIM = 64
_PITCH = EMO_DIM + 1
BATCH = 16384

_NC = 2
_NS = 16
_L = 16
_NW = _NC * _NS
_BPW = BATCH // _NW
_JPW = _BPW // 128
_TD = EMO_DIM // 8

_mesh = plsc.VectorSubcoreMesh(core_axis_name="c", subcore_axis_name="s")

_cp = pltpu.CompilerParams()
if "needs_layout_passes" in pltpu.CompilerParams.__dataclass_fields__:
    _cp = dataclasses.replace(_cp, needs_layout_passes=False)
if "use_tc_tiling_on_sc" in pltpu.CompilerParams.__dataclass_fields__:
    _cp = dataclasses.replace(_cp, use_tc_tiling_on_sc=False)


@jax.jit
def _emotion_encode(emo_id, strength, table_flat):
    @functools.partial(
        pl.kernel,
        out_type=jax.ShapeDtypeStruct((BATCH // 2, 2 * EMO_DIM), jnp.float32),
        mesh=_mesh,
        compiler_params=_cp,
        scratch_types=[
            pltpu.VMEM_SHARED((NUM_EMOTIONS * _PITCH,), jnp.float32),
            pltpu.VMEM((NUM_EMOTIONS * _PITCH,), jnp.float32),
            pltpu.VMEM((_BPW,), jnp.int32),
            pltpu.VMEM((_BPW,), jnp.float32),
        ]
        + [pltpu.VMEM((EMO_DIM, 128), jnp.float32) for _ in range(2)]
        + [pltpu.SemaphoreType.DMA, pltpu.SemaphoreType.DMA],
    )
    def k(emo_hbm, str_hbm, tab_hbm, out_hbm, tab_s, tab_v, idx_v, str_v,
          t0, t1, sem_t, sem_o):
        sid = lax.axis_index("s")
        wid = sid * _NC + lax.axis_index("c")
        base = wid * _BPW

        @pl.when(sid == 0)
        def _():
            pltpu.sync_copy(tab_hbm, tab_s)

        pltpu.sync_copy(emo_hbm.at[pl.ds(base, _BPW)], idx_v)
        pltpu.sync_copy(str_hbm.at[pl.ds(base, _BPW)], str_v)
        plsc.subcore_barrier()
        pltpu.sync_copy(tab_s, tab_v)

        @pl.loop(0, _JPW // 2)
        def _(p):
            for half, trows_v in enumerate((t0, t1)):
                jl = 2 * p + half

                @pl.when(p > 0)
                def _(trows_v=trows_v):
                    for i in range(_TD):
                        pltpu.make_async_copy(
                            trows_v.at[pl.ds(8 * i, 8)],
                            out_hbm.at[pl.ds(i * 1024, 8)],
                            sem_o,
                        ).wait()

                @plsc.parallel_loop(0, 128 // _L)
                def _(bg, jl=jl, trows_v=trows_v):
                    o = jl * 128 + bg * _L
                    e = idx_v[pl.ds(o, _L)]
                    s = str_v[pl.ds(o, _L)]
                    ebase = e * _PITCH

                    @plsc.parallel_loop(0, EMO_DIM, unroll=8)
                    def _(d):
                        v = plsc.load_gather(tab_v, [ebase + d])
                        trows_v[d, pl.ds(bg * _L, _L)] = v * s

                jg = wid * _JPW + jl
                for i in range(_TD):
                    pltpu.async_copy(
                        trows_v.at[pl.ds(8 * i, 8)],
                        out_hbm.at[pl.ds((i * 128 + jg) * 8, 8)],
                        sem_o,
                    )

        for trows_v in (t0, t1):
            for i in range(_TD):
                pltpu.make_async_copy(
                    trows_v.at[pl.ds(8 * i, 8)],
                    out_hbm.at[pl.ds(i * 1024, 8)],
                    sem_o,
                ).wait()

    return k(emo_id, strength, table_flat)


def kernel(emo_id, strength, table):
    tab65 = jnp.pad(table, ((0, 0), (0, _PITCH - EMO_DIM))).reshape(-1)
    w = _emotion_encode(emo_id.astype(jnp.int32), strength, tab65)
    return (
        w.reshape(_TD, 128, 8, 128)
        .transpose(1, 3, 0, 2)
        .reshape(BATCH, EMO_DIM)
    )
